# Initial kernel scaffold; baseline (speedup 1.0000x reference)
import functools

import jax
import jax.numpy as jnp
from jax import lax
from jax.experimental import pallas as pl
from jax.experimental.pallas import tpu as pltpu
from jax.experimental.pallas import tpu_sc as plsc

N_NODES = 10000
D = 128
N_PAD = 10240
NC = 2
NS = 16
NW = NC * NS
BATCH = 128
ROWS_PT = N_PAD // NS


def _mesh():
    return plsc.VectorSubcoreMesh(core_axis_name="c", subcore_axis_name="s")


def _deg_body(nbatch, dst_hbm, zeros_hbm, ones_hbm, deg_hbm, acc, dst_v, ones_v):
    cid = lax.axis_index("c")
    sid = lax.axis_index("s")
    wid = sid * NC + cid
    pltpu.sync_copy(zeros_hbm.at[pl.ds(sid * ROWS_PT, ROWS_PT)],
                    acc.at[pl.ds(sid * ROWS_PT, ROWS_PT)])
    pltpu.sync_copy(dst_hbm.at[wid], dst_v)
    pltpu.sync_copy(ones_hbm, ones_v)
    plsc.subcore_barrier()

    def body(b, carry):
        pltpu.sync_copy(ones_v, acc.at[dst_v.at[b]], add=True)
        return carry

    lax.fori_loop(0, nbatch, body, 0)
    plsc.subcore_barrier()
    off = cid * N_PAD + sid * ROWS_PT
    pltpu.sync_copy(acc.at[pl.ds(sid * ROWS_PT, ROWS_PT)],
                    deg_hbm.at[pl.ds(off, ROWS_PT)])


def _deg_call(dst3, zeros16, ones16):
    nbatch = dst3.shape[1]
    k = pl.kernel(
        functools.partial(_deg_body, nbatch),
        out_type=jax.ShapeDtypeStruct((NC * N_PAD, 16), jnp.float32),
        mesh=_mesh(),
        scratch_types=[
            pltpu.VMEM_SHARED((N_PAD, 16), jnp.float32),
            pltpu.VMEM((nbatch, BATCH), jnp.int32),
            pltpu.VMEM((BATCH, 16), jnp.float32),
        ],
    )
    return k(dst3, zeros16, ones16)


def _agg_body(nbatch, h_hbm, src_hbm, dst_hbm, zeros_hbm, out_hbm,
              acc, src_v, dst_v, rows_v, gsem):
    cid = lax.axis_index("c")
    sid = lax.axis_index("s")
    wid = sid * NC + cid
    pltpu.sync_copy(zeros_hbm.at[pl.ds(sid * ROWS_PT, ROWS_PT)],
                    acc.at[pl.ds(sid * ROWS_PT, ROWS_PT)])
    pltpu.sync_copy(src_hbm.at[wid], src_v)
    pltpu.sync_copy(dst_hbm.at[wid], dst_v)
    plsc.subcore_barrier()

    def body(b, carry):
        pltpu.async_copy(h_hbm.at[src_v.at[b]], rows_v, gsem).wait()
        pltpu.sync_copy(rows_v, acc.at[dst_v.at[b]], add=True)
        return carry

    lax.fori_loop(0, nbatch, body, 0)
    plsc.subcore_barrier()
    off = cid * N_PAD + sid * ROWS_PT
    pltpu.sync_copy(acc.at[pl.ds(sid * ROWS_PT, ROWS_PT)],
                    out_hbm.at[pl.ds(off, ROWS_PT)])


def _agg_call(hp, src3, dst3, zeros128):
    nbatch = src3.shape[1]
    k = pl.kernel(
        functools.partial(_agg_body, nbatch),
        out_type=jax.ShapeDtypeStruct((NC * N_PAD, D), jnp.float32),
        mesh=_mesh(),
        scratch_types=[
            pltpu.VMEM_SHARED((N_PAD, D), jnp.float32),
            pltpu.VMEM((nbatch, BATCH), jnp.int32),
            pltpu.VMEM((nbatch, BATCH), jnp.int32),
            pltpu.VMEM((BATCH, D), jnp.float32),
            pltpu.SemaphoreType.DMA,
        ],
    )
    return k(hp, src3, dst3, zeros128)


_R = 1024


def _dinv_from(deg_ref):
    deg = deg_ref[0, :, 0:1] + deg_ref[1, :, 0:1] + 1.0
    return lax.rsqrt(deg)


def _mm1_body(deg_ref, x_ref, w_ref, o_ref):
    dinv = _dinv_from(deg_ref)
    h = jnp.dot(x_ref[...], w_ref[...], preferred_element_type=jnp.float32)
    o_ref[...] = h * dinv


def _ln_gelu(s, g_ref, be_ref):
    m = jnp.mean(s, axis=-1, keepdims=True)
    v = jnp.mean((s - m) ** 2, axis=-1, keepdims=True)
    z = (s - m) * lax.rsqrt(v + 1e-5) * g_ref[...] + be_ref[...]
    return jax.nn.gelu(z, approximate=False)


def _mid_body(deg_ref, acc_ref, h_ref, b_ref, g_ref, be_ref, w_ref, o_ref):
    dinv = _dinv_from(deg_ref)
    s = (acc_ref[0] + acc_ref[1] + h_ref[...]) * dinv + b_ref[...]
    z = _ln_gelu(s, g_ref, be_ref)
    h2 = jnp.dot(z, w_ref[...], preferred_element_type=jnp.float32)
    o_ref[...] = h2 * dinv


def _fin_body(deg_ref, acc_ref, h_ref, b_ref, g_ref, be_ref, o_ref):
    dinv = _dinv_from(deg_ref)
    s = (acc_ref[0] + acc_ref[1] + h_ref[...]) * dinv + b_ref[...]
    o_ref[...] = _ln_gelu(s, g_ref, be_ref)


_deg_spec = pl.BlockSpec((2, _R, 16), lambda i: (0, i, 0))
_acc_spec = pl.BlockSpec((2, _R, D), lambda i: (0, i, 0))
_row_spec = pl.BlockSpec((_R, D), lambda i: (i, 0))
_vec_spec = pl.BlockSpec((1, D), lambda i: (0, 0))
_w_spec = pl.BlockSpec((D, D), lambda i: (0, 0))
_grid = (N_PAD // _R,)


def _mm1_call(deg, x_pad, W):
    return pl.pallas_call(
        _mm1_body,
        grid=_grid,
        in_specs=[_deg_spec, _row_spec, _w_spec],
        out_specs=_row_spec,
        out_shape=jax.ShapeDtypeStruct((N_PAD, D), jnp.float32),
    )(deg, x_pad, W)


def _mid_call(deg, acc, hp, b, g, be, W):
    return pl.pallas_call(
        _mid_body,
        grid=_grid,
        in_specs=[_deg_spec, _acc_spec, _row_spec, _vec_spec, _vec_spec,
                  _vec_spec, _w_spec],
        out_specs=_row_spec,
        out_shape=jax.ShapeDtypeStruct((N_PAD, D), jnp.float32),
    )(deg, acc, hp, b, g, be, W)


def _fin_call(deg, acc, hp, b, g, be):
    return pl.pallas_call(
        _fin_body,
        grid=_grid,
        in_specs=[_deg_spec, _acc_spec, _row_spec, _vec_spec, _vec_spec,
                  _vec_spec],
        out_specs=_row_spec,
        out_shape=jax.ShapeDtypeStruct((N_PAD, D), jnp.float32),
    )(deg, acc, hp, b, g, be)


def kernel(x, edge_index, W1, b1, g1, beta1, W2, b2, g2, beta2):
    n = x.shape[0]
    e = edge_index.shape[1]
    e_pad = NW * BATCH * ((e + NW * BATCH - 1) // (NW * BATCH))
    nbatch = e_pad // (NW * BATCH)

    x_pad = jnp.zeros((N_PAD, D), x.dtype).at[:n].set(x)
    pad = jnp.full((e_pad - e,), n, dtype=jnp.int32)
    src3 = jnp.concatenate([edge_index[0], pad]).reshape(NW, nbatch, BATCH)
    dst3 = jnp.concatenate([edge_index[1], pad]).reshape(NW, nbatch, BATCH)

    zeros128 = jnp.zeros((N_PAD, D), jnp.float32)
    zeros16 = jnp.zeros((N_PAD, 16), jnp.float32)
    ones16 = jnp.ones((BATCH, 16), jnp.float32)

    degpart = _deg_call(dst3, zeros16, ones16).reshape(NC, N_PAD, 16)

    b1r, g1r, be1r = b1.reshape(1, D), g1.reshape(1, D), beta1.reshape(1, D)
    b2r, g2r, be2r = b2.reshape(1, D), g2.reshape(1, D), beta2.reshape(1, D)

    h1p = _mm1_call(degpart, x_pad, W1)
    acc1 = _agg_call(h1p, src3, dst3, zeros128).reshape(NC, N_PAD, D)
    h2p = _mid_call(degpart, acc1, h1p, b1r, g1r, be1r, W2)
    acc2 = _agg_call(h2p, src3, dst3, zeros128).reshape(NC, N_PAD, D)
    out = _fin_call(degpart, acc2, h2p, b2r, g2r, be2r)
    return out[:n]


# speedup vs baseline: 9.3311x; 9.3311x over previous
import functools

import jax
import jax.numpy as jnp
from jax import lax
from jax.experimental import pallas as pl
from jax.experimental.pallas import tpu as pltpu
from jax.experimental.pallas import tpu_sc as plsc

N_NODES = 10000
D = 128
N_PAD = 10240
NC = 2
NS = 16
NW = NC * NS
BATCH = 128
ROWS_PT = N_PAD // NS


def _mesh():
    return plsc.VectorSubcoreMesh(core_axis_name="c", subcore_axis_name="s",
                                  num_cores=NC, num_subcores=NS)


def _agg_body(nbatch, h_hbm, src_hbm, dst_hbm, zeros_hbm, out_hbm,
              acc, src_v, dst_v, rows_v, gsem):
    cid = lax.axis_index("c")
    sid = lax.axis_index("s")
    wid = sid * NC + cid
    pltpu.sync_copy(zeros_hbm.at[pl.ds(sid * ROWS_PT, ROWS_PT)],
                    acc.at[pl.ds(sid * ROWS_PT, ROWS_PT)])
    pltpu.sync_copy(src_hbm.at[wid], src_v)
    pltpu.sync_copy(dst_hbm.at[wid], dst_v)
    plsc.subcore_barrier()

    def body(b, carry):
        pltpu.async_copy(h_hbm.at[src_v.at[b]], rows_v, gsem).wait()
        pltpu.sync_copy(rows_v, acc.at[dst_v.at[b]], add=True)
        return carry

    lax.fori_loop(0, nbatch, body, 0)
    plsc.subcore_barrier()
    off = cid * N_PAD + sid * ROWS_PT
    pltpu.sync_copy(acc.at[pl.ds(sid * ROWS_PT, ROWS_PT)],
                    out_hbm.at[pl.ds(off, ROWS_PT)])


def _agg_call(hp, src3, dst3, zeros128):
    nbatch = src3.shape[1]
    k = pl.kernel(
        functools.partial(_agg_body, nbatch),
        out_type=jax.ShapeDtypeStruct((NC * N_PAD, D), jnp.float32),
        mesh=_mesh(),
        scratch_types=[
            pltpu.VMEM_SHARED((N_PAD, D), jnp.float32),
            pltpu.VMEM((nbatch, BATCH), jnp.int32),
            pltpu.VMEM((nbatch, BATCH), jnp.int32),
            pltpu.VMEM((BATCH, D), jnp.float32),
            pltpu.SemaphoreType.DMA,
        ],
    )
    return k(hp, src3, dst3, zeros128)


_R = 1024


def _dinv_from(deg_ref):
    deg = deg_ref[0, :, 0:1] + deg_ref[1, :, 0:1] + 1.0
    return lax.rsqrt(deg)


def _mm1_body(deg_ref, x_ref, w_ref, o_ref):
    dinv = _dinv_from(deg_ref)
    h = jnp.dot(x_ref[...], w_ref[...], preferred_element_type=jnp.float32)
    o_ref[...] = h * dinv


def _ln_gelu(s, g_ref, be_ref):
    m = jnp.mean(s, axis=-1, keepdims=True)
    v = jnp.mean((s - m) ** 2, axis=-1, keepdims=True)
    z = (s - m) * lax.rsqrt(v + 1e-5) * g_ref[...] + be_ref[...]
    return 0.5 * z * (1.0 + lax.erf(z * 0.7071067811865476))


def _mid_body(deg_ref, acc_ref, h_ref, b_ref, g_ref, be_ref, w_ref, o_ref):
    dinv = _dinv_from(deg_ref)
    s = (acc_ref[0] + acc_ref[1] + h_ref[...]) * dinv + b_ref[...]
    z = _ln_gelu(s, g_ref, be_ref)
    h2 = jnp.dot(z, w_ref[...], preferred_element_type=jnp.float32)
    o_ref[...] = h2 * dinv


def _fin_body(deg_ref, acc_ref, h_ref, b_ref, g_ref, be_ref, o_ref):
    dinv = _dinv_from(deg_ref)
    s = (acc_ref[0] + acc_ref[1] + h_ref[...]) * dinv + b_ref[...]
    o_ref[...] = _ln_gelu(s, g_ref, be_ref)


_deg_spec = pl.BlockSpec((2, _R, D), lambda i: (0, i, 0))
_acc_spec = pl.BlockSpec((2, _R, D), lambda i: (0, i, 0))
_row_spec = pl.BlockSpec((_R, D), lambda i: (i, 0))
_vec_spec = pl.BlockSpec((1, D), lambda i: (0, 0))
_w_spec = pl.BlockSpec((D, D), lambda i: (0, 0))
_grid = (N_PAD // _R,)


def _mm1_call(deg, x_pad, W):
    return pl.pallas_call(
        _mm1_body,
        grid=_grid,
        in_specs=[_deg_spec, _row_spec, _w_spec],
        out_specs=_row_spec,
        out_shape=jax.ShapeDtypeStruct((N_PAD, D), jnp.float32),
    )(deg, x_pad, W)


def _mid_call(deg, acc, hp, b, g, be, W):
    return pl.pallas_call(
        _mid_body,
        grid=_grid,
        in_specs=[_deg_spec, _acc_spec, _row_spec, _vec_spec, _vec_spec,
                  _vec_spec, _w_spec],
        out_specs=_row_spec,
        out_shape=jax.ShapeDtypeStruct((N_PAD, D), jnp.float32),
    )(deg, acc, hp, b, g, be, W)


def _fin_call(deg, acc, hp, b, g, be):
    return pl.pallas_call(
        _fin_body,
        grid=_grid,
        in_specs=[_deg_spec, _acc_spec, _row_spec, _vec_spec, _vec_spec,
                  _vec_spec],
        out_specs=_row_spec,
        out_shape=jax.ShapeDtypeStruct((N_PAD, D), jnp.float32),
    )(deg, acc, hp, b, g, be)


def kernel(x, edge_index, W1, b1, g1, beta1, W2, b2, g2, beta2):
    n = x.shape[0]
    e = edge_index.shape[1]
    e_pad = NW * BATCH * ((e + NW * BATCH - 1) // (NW * BATCH))
    nbatch = e_pad // (NW * BATCH)

    x_pad = jnp.zeros((N_PAD, D), x.dtype).at[:n].set(x)
    pad = jnp.full((e_pad - e,), n, dtype=jnp.int32)
    src3 = jnp.concatenate([edge_index[0], pad]).reshape(NW, nbatch, BATCH)
    dst3 = jnp.concatenate([edge_index[1], pad]).reshape(NW, nbatch, BATCH)

    zeros128 = jnp.zeros((N_PAD, D), jnp.float32)
    ones128 = jnp.ones((N_PAD, D), jnp.float32)

    degpart = _agg_call(ones128, src3, dst3, zeros128).reshape(NC, N_PAD, D)

    b1r, g1r, be1r = b1.reshape(1, D), g1.reshape(1, D), beta1.reshape(1, D)
    b2r, g2r, be2r = b2.reshape(1, D), g2.reshape(1, D), beta2.reshape(1, D)

    h1p = _mm1_call(degpart, x_pad, W1)
    acc1 = _agg_call(h1p, src3, dst3, zeros128).reshape(NC, N_PAD, D)
    h2p = _mid_call(degpart, acc1, h1p, b1r, g1r, be1r, W2)
    acc2 = _agg_call(h2p, src3, dst3, zeros128).reshape(NC, N_PAD, D)
    out = _fin_call(degpart, acc2, h2p, b2r, g2r, be2r)
    return out[:n]
